# Initial kernel scaffold; baseline (speedup 1.0000x reference)
import jax
import jax.numpy as jnp
from jax.experimental import pallas as pl


def kernel(x, table):
    raise NotImplementedError("write your pallas kernel here")


# speedup vs baseline: 1.0937x; 1.0937x over previous
import functools

import jax
import jax.numpy as jnp
from jax import lax
from jax.experimental import pallas as pl
from jax.experimental.pallas import tpu as pltpu
from jax.experimental.pallas import tpu_sc as plsc


def _gather_rows(flat_idx, table, chunk):
    (b_total,) = flat_idx.shape
    _, emb = table.shape
    info = plsc.get_sparse_core_info()
    num_workers = info.num_cores * info.num_subcores
    per_worker = b_total // num_workers
    n_chunks = per_worker // chunk
    assert per_worker % chunk == 0 and b_total % num_workers == 0

    mesh = plsc.VectorSubcoreMesh(core_axis_name="c", subcore_axis_name="s")

    @functools.partial(
        pl.kernel,
        mesh=mesh,
        out_type=jax.ShapeDtypeStruct((b_total, emb), jnp.float32),
        scratch_types=[
            pltpu.VMEM((chunk,), jnp.int32),
            pltpu.VMEM((chunk, emb), jnp.float32),
            pltpu.SemaphoreType.DMA,
        ],
        compiler_params=pltpu.CompilerParams(use_tc_tiling_on_sc=False),
    )
    def k(idx_hbm, table_hbm, out_hbm, idx_v, rows_v, sem):
        wid = lax.axis_index("s") * info.num_cores + lax.axis_index("c")
        base_w = wid * per_worker

        def body(i, carry):
            base = base_w + i * chunk
            pltpu.sync_copy(idx_hbm.at[pl.ds(base, chunk)], idx_v)
            pltpu.async_copy(table_hbm.at[idx_v], rows_v, sem).wait()
            pltpu.sync_copy(rows_v, out_hbm.at[pl.ds(base, chunk)])
            return carry

        lax.fori_loop(0, n_chunks, body, 0)

    return k(flat_idx, table)


def kernel(x, table):
    batch, hist = x.shape
    _, emb = table.shape
    flat = x.reshape(batch * hist)
    out = _gather_rows(flat, table, chunk=1024)
    return out.reshape(batch, hist, emb)


# speedup vs baseline: 1.1127x; 1.0174x over previous
import functools

import jax
import jax.numpy as jnp
from jax import lax
from jax.experimental import pallas as pl
from jax.experimental.pallas import tpu as pltpu
from jax.experimental.pallas import tpu_sc as plsc


def _gather_rows(flat_idx, table, chunk, nbuf):
    (b_total,) = flat_idx.shape
    _, emb = table.shape
    info = plsc.get_sparse_core_info()
    num_workers = info.num_cores * info.num_subcores
    per_worker = b_total // num_workers
    n_chunks = per_worker // chunk
    n_groups = n_chunks // nbuf
    assert b_total % num_workers == 0
    assert per_worker % chunk == 0 and n_chunks % nbuf == 0
    assert chunk % 8 == 0

    mesh = plsc.VectorSubcoreMesh(core_axis_name="c", subcore_axis_name="s")

    @functools.partial(
        pl.kernel,
        mesh=mesh,
        out_type=jax.ShapeDtypeStruct((b_total, emb), jnp.float32),
        scratch_types=(
            [pltpu.VMEM((per_worker,), jnp.int32)]
            + [pltpu.VMEM((chunk, emb), jnp.float32) for _ in range(nbuf)]
            + [pltpu.SemaphoreType.DMA for _ in range(2 * nbuf)]
        ),
        compiler_params=pltpu.CompilerParams(use_tc_tiling_on_sc=False),
    )
    def k(idx_hbm, table_hbm, out_hbm, idx_all, *bufs):
        rows = bufs[:nbuf]
        gsem = bufs[nbuf : 2 * nbuf]
        ssem = bufs[2 * nbuf :]
        wid = lax.axis_index("s") * info.num_cores + lax.axis_index("c")
        base_w = wid * per_worker
        pltpu.sync_copy(idx_hbm.at[pl.ds(base_w, per_worker)], idx_all)

        def gather(g, b):
            return pltpu.make_async_copy(
                table_hbm.at[idx_all.at[pl.ds(g * chunk, chunk)]],
                rows[b],
                gsem[b],
            )

        def store(g, b):
            return pltpu.make_async_copy(
                rows[b],
                out_hbm.at[pl.ds(base_w + g * chunk, chunk)],
                ssem[b],
            )

        for b in range(nbuf):
            gather(b, b).start()

        def body(j, carry):
            for b in range(nbuf):
                g = j * nbuf + b
                gather(g, b).wait()
                store(g, b).start()
                store(g, b).wait()
                gather(g + nbuf, b).start()
            return carry

        lax.fori_loop(0, n_groups - 1, body, 0)

        last = (n_groups - 1) * nbuf
        for b in range(nbuf):
            gather(last + b, b).wait()
            store(last + b, b).start()
        for b in range(nbuf):
            store(last + b, b).wait()

    return k(flat_idx, table)


def kernel(x, table):
    batch, hist = x.shape
    _, emb = table.shape
    flat = x.reshape(batch * hist)
    out = _gather_rows(flat, table, chunk=640, nbuf=4)
    return out.reshape(batch, hist, emb)
